# Initial kernel scaffold; baseline (speedup 1.0000x reference)
import jax
import jax.numpy as jnp
from jax.experimental import pallas as pl


def kernel(observed_mask, rand_unit, sample_ratios):
    raise NotImplementedError("write your pallas kernel here")


# speedup vs baseline: 9.0064x; 9.0064x over previous
import jax
import jax.numpy as jnp
from jax.experimental import pallas as pl

_ROWS_PER_BLOCK = 8
_ONE_F32_BITS = 0x3F800000


def _block_body(obs_ref, rnd_ref, ratio_ref, out_ref):
    obs = obs_ref[...]
    v = rnd_ref[...] * obs
    bits = jax.lax.bitcast_convert_type(v, jnp.int32)

    num_obs = jnp.sum(obs, axis=1)
    ratios = ratio_ref[0, 0]
    k = jnp.round(num_obs * ratios).astype(jnp.int32)

    lo0 = jnp.zeros_like(k)
    hi0 = jnp.full_like(k, _ONE_F32_BITS)

    def step(_, lohi):
        lo, hi = lohi
        mid = (lo + hi) >> 1
        cnt = jnp.sum((bits >= mid[:, None]).astype(jnp.int32), axis=1)
        ge = cnt >= k
        return jnp.where(ge, mid, lo), jnp.where(ge, hi, mid)

    lo, _ = jax.lax.fori_loop(0, 30, step, (lo0, hi0))
    t = lo[:, None]

    above = bits > t
    c_gt = jnp.sum(above.astype(jnp.int32), axis=1)
    r = k - c_gt
    is_tie = bits == t
    idx = jax.lax.broadcasted_iota(jnp.int32, is_tie.shape, 1)

    def tstep(_, lohi):
        lo, hi = lohi
        mid = (lo + hi) >> 1
        cnt = jnp.sum((is_tie & (idx < mid[:, None])).astype(jnp.int32),
                      axis=1)
        le = cnt <= r
        return jnp.where(le, mid, lo), jnp.where(le, hi, mid)

    tlo0 = jnp.zeros_like(k)
    thi0 = jnp.full_like(k, is_tie.shape[1] + 1)
    m, _ = jax.lax.fori_loop(0, 16, tstep, (tlo0, thi0))
    masked = above | (is_tie & (idx < m[:, None]))
    out_ref[...] = jnp.where((v > 0) & jnp.logical_not(masked),
                             jnp.float32(1.0), jnp.float32(0.0))


def kernel(observed_mask, rand_unit, sample_ratios):
    B, K, L = observed_mask.shape
    N = K * L
    R = _ROWS_PER_BLOCK
    grid = B // R

    obs2 = observed_mask.reshape(B, N)
    rnd2 = rand_unit.reshape(B, N)
    low, high = 0.1, 0.4
    ratios = low + (high - low) * sample_ratios
    ratios3 = ratios.reshape(grid, 1, R)

    out = pl.pallas_call(
        _block_body,
        grid=(grid,),
        in_specs=[
            pl.BlockSpec((R, N), lambda i: (i, 0)),
            pl.BlockSpec((R, N), lambda i: (i, 0)),
            pl.BlockSpec((1, 1, R), lambda i: (i, 0, 0)),
        ],
        out_specs=pl.BlockSpec((R, N), lambda i: (i, 0)),
        out_shape=jax.ShapeDtypeStruct((B, N), jnp.float32),
    )(obs2, rnd2, ratios3)
    return out.reshape(B, K, L)


# speedup vs baseline: 9.6535x; 1.0718x over previous
import jax
import jax.numpy as jnp
from jax import lax
from jax.experimental import pallas as pl
from jax.experimental.pallas import tpu as pltpu
from jax.experimental.pallas import tpu_sc as plsc

_B, _N = 128, 32768
_VECS = _N // 16
_ONE_F32_BITS = 0x3F800000
_NW = 32
_ROWS_PER_W = _B // _NW

def _sc_body(obs_hbm, rnd_hbm, ratio_hbm, out_hbm, vbuf, obuf, rbuf):
    _I0 = jnp.zeros((16,), jnp.int32)
    cid = lax.axis_index("c")
    sid = lax.axis_index("s")
    wid = sid * 2 + cid
    base = wid * _ROWS_PER_W

    def row_body(j, carry):
        row = base + j
        pltpu.sync_copy(obs_hbm.at[row], obuf)
        pltpu.sync_copy(rnd_hbm.at[row], vbuf)
        pltpu.sync_copy(ratio_hbm.at[row], rbuf)

        ratio = rbuf[...]

        def p0(i, acc):
            o = obuf[pl.ds(i * 16, 16)]
            r = vbuf[pl.ds(i * 16, 16)]
            vbuf[pl.ds(i * 16, 16)] = r * o
            return acc + plsc.all_reduce_population_count(o > jnp.float32(0))

        nobs = lax.fori_loop(0, _VECS, p0, _I0, unroll=8)

        x = nobs.astype(jnp.float32) * ratio
        xi = x.astype(jnp.int32)
        frac = x - xi.astype(jnp.float32)
        k = xi + jnp.where(frac > jnp.float32(0.5), 1,
                           jnp.where(frac == jnp.float32(0.5), xi & 1, 0))

        def search(_, st):
            lo, hi, c_hi = st
            mid = (lo + hi) >> 1

            def inner(i, a):
                b = plsc.bitcast(vbuf[pl.ds(i * 16, 16)], jnp.int32)
                return a + plsc.all_reduce_population_count(b >= mid)

            cnt = lax.fori_loop(0, _VECS, inner, _I0, unroll=8)
            ge = cnt >= k
            return (jnp.where(ge, mid, lo), jnp.where(ge, hi, mid),
                    jnp.where(ge, c_hi, cnt))

        tvec, _, c_gt = lax.fori_loop(
            0, 30, search,
            (_I0, jnp.full((16,), _ONE_F32_BITS, jnp.int32), _I0))

        rvec = k - c_gt

        def outp(i, cr):
            v = vbuf[pl.ds(i * 16, 16)]
            b = plsc.bitcast(v, jnp.int32)
            m = b == tvec
            within = plsc.cumsum(m.astype(jnp.int32))
            tie_mask = m & ((within + cr) <= rvec)
            keep = (v > jnp.float32(0.0)) & jnp.logical_not(
                (b > tvec) | tie_mask)
            obuf[pl.ds(i * 16, 16)] = jnp.where(
                keep, jnp.float32(1.0), jnp.float32(0.0))
            return cr + plsc.all_reduce_population_count(m)

        lax.fori_loop(0, _VECS, outp, _I0, unroll=4)

        pltpu.sync_copy(obuf, out_hbm.at[row])
        return carry

    lax.fori_loop(0, _ROWS_PER_W, row_body, jnp.int32(0))


def kernel(observed_mask, rand_unit, sample_ratios):
    B, K, L = observed_mask.shape
    N = K * L
    obs2 = observed_mask.reshape(B, N)
    rnd2 = rand_unit.reshape(B, N)
    low, high = 0.1, 0.4
    ratios = low + (high - low) * sample_ratios

    mesh = plsc.VectorSubcoreMesh(core_axis_name="c", subcore_axis_name="s",
                                  num_cores=2, num_subcores=16)
    run = pl.kernel(
        _sc_body,
        out_type=jax.ShapeDtypeStruct((B, N), jnp.float32),
        mesh=mesh,
        compiler_params=pltpu.CompilerParams(needs_layout_passes=False),
        scratch_types=[
            pltpu.VMEM((N,), jnp.float32),
            pltpu.VMEM((N,), jnp.float32),
            pltpu.VMEM((16,), jnp.float32),
        ],
    )
    ratios16 = jnp.broadcast_to(ratios[:, None], (B, 16))
    out = run(obs2, rnd2, ratios16)
    return out.reshape(B, K, L)
